# baseline (device time: 34167 ns/iter reference)
import jax
import jax.numpy as jnp
from jax import lax
from jax.experimental import pallas as pl
from jax.experimental.pallas import tpu as pltpu

N_DEV = 4
B_PER = 2
SQ = 128
D = 512
H_LOC = 8
DH = 64
ROWS = B_PER * SQ


def kernel(x, Wq, Wo, K_ext, V_ext):
    B, Skv = K_ext.shape[0], K_ext.shape[1]
    K2 = K_ext.reshape(B, Skv, K_ext.shape[2] * K_ext.shape[3])
    V2 = V_ext.reshape(B, Skv, V_ext.shape[2] * V_ext.shape[3])
    me = lax.axis_index("i").astype(jnp.int32)
    me_arr = me.reshape((1,))

    def body(me_ref, x_ref, wq_ref, wo_ref, k_ref, v_ref, out_ref,
             x_stage, xr, ps, pr, kbuf, vbuf,
             xssem, xrsem, pssem, prsem, kvsem):
        my = me_ref[0]
        others = [lax.rem(my + d, N_DEV) for d in (1, 2, 3)]

        barrier = pltpu.get_barrier_semaphore()
        for tgt in others:
            pl.semaphore_signal(
                barrier, inc=1, device_id=(tgt,),
                device_id_type=pl.DeviceIdType.MESH,
            )
        pl.semaphore_wait(barrier, 3)

        xbf = x_ref[...].reshape(ROWS, D).astype(jnp.bfloat16)
        x_stage[...] = xbf
        xsends = []
        for d in (2, 1, 3):
            r = pltpu.make_async_remote_copy(
                src_ref=x_stage,
                dst_ref=xr.at[d - 1],
                send_sem=xssem.at[d - 1],
                recv_sem=xrsem.at[d - 1],
                device_id=(others[d - 1],),
                device_id_type=pl.DeviceIdType.MESH,
            )
            r.start()
            xsends.append(r)

        chunk_owner = [my, others[2], others[0], others[1]]
        kv_dmas = []
        for c in range(N_DEV):
            dmas = []
            bg0 = chunk_owner[c] * B_PER
            for src, dst in ((k_ref, kbuf), (v_ref, vbuf)):
                dma = pltpu.make_async_copy(
                    src.at[pl.ds(bg0, B_PER), :, pl.ds(my * D, D)],
                    dst.at[c],
                    kvsem.at[c],
                )
                dma.start()
                dmas.append(dma)
            kv_dmas.append(dmas)

        wq = (wq_ref[...] * 0.125).astype(jnp.bfloat16)
        wo = wo_ref[...].astype(jnp.bfloat16)

        def partial_half(xc, c, bi, qc):
            kb = kbuf[c, bi].astype(jnp.bfloat16)
            vb = vbuf[c, bi].astype(jnp.bfloat16)
            k3 = jnp.transpose(kb.reshape(Skv, H_LOC, DH), (1, 0, 2))
            v3 = jnp.transpose(vb.reshape(Skv, H_LOC, DH), (1, 0, 2))
            q3 = jnp.transpose(
                qc[bi * SQ:(bi + 1) * SQ].reshape(SQ, H_LOC, DH),
                (1, 0, 2))
            s = lax.dot_general(
                q3, k3, (((2,), (2,)), ((0,), (0,))),
                preferred_element_type=jnp.float32,
            )
            p = jnp.exp(s.astype(jnp.bfloat16))
            l = jnp.sum(p, axis=2, keepdims=True,
                        dtype=jnp.float32)
            o3 = lax.dot_general(
                p, v3,
                (((2,), (1,)), ((0,), (0,))),
                preferred_element_type=jnp.float32,
            ) * (1.0 / l)
            ao = jnp.transpose(o3, (1, 0, 2)).reshape(SQ, H_LOC * DH)
            return jnp.dot(ao.astype(jnp.bfloat16), wo,
                           preferred_element_type=jnp.float32)

        def qc_for(xc, c):
            for dma in kv_dmas[c]:
                dma.wait()
            return jnp.dot(
                xc, wq, preferred_element_type=jnp.float32
            ).astype(jnp.bfloat16)

        qc0 = qc_for(xbf, 0)
        p_own = [partial_half(xbf, 0, bi, qc0) for bi in range(B_PER)]

        psends = []
        for s in (0, 2, 1):
            rcv = pltpu.make_async_remote_copy(
                src_ref=x_stage,
                dst_ref=xr.at[s],
                send_sem=xssem.at[s],
                recv_sem=xrsem.at[s],
                device_id=(my,),
                device_id_type=pl.DeviceIdType.MESH,
            )
            rcv.wait_recv()
            d2 = 3 - s
            owner = others[d2 - 1]
            c = {0: 1, 2: 2, 1: 3}[s]
            qc = qc_for(xr[s], c)
            for bi in range(B_PER):
                pv = partial_half(xr[s], c, bi, qc)
                ps[d2 - 1, pl.ds(bi * SQ, SQ)] = pv.astype(jnp.bfloat16)
                r2 = pltpu.make_async_remote_copy(
                    src_ref=ps.at[d2 - 1, pl.ds(bi * SQ, SQ)],
                    dst_ref=pr.at[d2 - 1, pl.ds(bi * SQ, SQ)],
                    send_sem=pssem.at[d2 - 1, bi],
                    recv_sem=prsem.at[d2 - 1, bi],
                    device_id=(owner,),
                    device_id_type=pl.DeviceIdType.MESH,
                )
                r2.start()
                psends.append(r2)

        acc = jnp.concatenate(p_own, axis=0)
        for s in range(3):
            for bi in range(B_PER):
                rcv2 = pltpu.make_async_remote_copy(
                    src_ref=ps.at[s, pl.ds(bi * SQ, SQ)],
                    dst_ref=pr.at[s, pl.ds(bi * SQ, SQ)],
                    send_sem=pssem.at[s, bi],
                    recv_sem=prsem.at[s, bi],
                    device_id=(my,),
                    device_id_type=pl.DeviceIdType.MESH,
                )
                rcv2.wait_recv()
            acc = acc + pr[s].astype(jnp.float32)
        out_ref[...] = acc.astype(jnp.bfloat16).reshape(B_PER, SQ, D)
        for r in xsends + psends:
            r.wait_send()

    grid_spec = pltpu.PrefetchScalarGridSpec(
        num_scalar_prefetch=1,
        grid=(1,),
        in_specs=[
            pl.BlockSpec((B_PER, SQ, D), lambda i, m: (0, 0, 0)),
            pl.BlockSpec((D, D), lambda i, m: (0, 0)),
            pl.BlockSpec((D, D), lambda i, m: (0, 0)),
            pl.BlockSpec(memory_space=pl.ANY),
            pl.BlockSpec(memory_space=pl.ANY),
        ],
        out_specs=pl.BlockSpec((B_PER, SQ, D), lambda i, m: (0, 0, 0)),
        scratch_shapes=[
            pltpu.VMEM((ROWS, D), jnp.bfloat16),
            pltpu.VMEM((N_DEV - 1, ROWS, D), jnp.bfloat16),
            pltpu.VMEM((N_DEV - 1, ROWS, D), jnp.bfloat16),
            pltpu.VMEM((N_DEV - 1, ROWS, D), jnp.bfloat16),
            pltpu.VMEM((N_DEV, B_PER, Skv, D), jnp.float32),
            pltpu.VMEM((N_DEV, B_PER, Skv, D), jnp.float32),
            pltpu.SemaphoreType.DMA((N_DEV - 1,)),
            pltpu.SemaphoreType.DMA((N_DEV - 1,)),
            pltpu.SemaphoreType.DMA((N_DEV - 1, B_PER)),
            pltpu.SemaphoreType.DMA((N_DEV - 1, B_PER)),
            pltpu.SemaphoreType.DMA((N_DEV,)),
        ],
    )

    return pl.pallas_call(
        body,
        out_shape=jax.ShapeDtypeStruct((B_PER, SQ, D), jnp.bfloat16),
        grid_spec=grid_spec,
        compiler_params=pltpu.CompilerParams(collective_id=0),
    )(me_arr, x, Wq, Wo, K2, V2)


# device time: 31872 ns/iter; 1.0720x vs baseline; 1.0720x over previous
import jax
import jax.numpy as jnp
from jax import lax
from jax.experimental import pallas as pl
from jax.experimental.pallas import tpu as pltpu

N_DEV = 4
B_PER = 2
SQ = 128
D = 512
H_LOC = 8
DH = 64
ROWS = B_PER * SQ


def kernel(x, Wq, Wo, K_ext, V_ext):
    B, Skv = K_ext.shape[0], K_ext.shape[1]
    K2 = K_ext.reshape(B, Skv, K_ext.shape[2] * K_ext.shape[3])
    V2 = V_ext.reshape(B, Skv, V_ext.shape[2] * V_ext.shape[3])
    me = lax.axis_index("i").astype(jnp.int32)
    me_arr = me.reshape((1,))

    def body(me_ref, x_ref, wq_ref, wo_ref, k_ref, v_ref, out_ref,
             x_stage, xr, ps, pr, kbuf, vbuf,
             xssem, xrsem, pssem, prsem, kvsem):
        my = me_ref[0]
        others = [lax.rem(my + d, N_DEV) for d in (1, 2, 3)]

        barrier = pltpu.get_barrier_semaphore()
        for tgt in others:
            pl.semaphore_signal(
                barrier, inc=1, device_id=(tgt,),
                device_id_type=pl.DeviceIdType.MESH,
            )
        pl.semaphore_wait(barrier, 3)

        xbf = x_ref[...].reshape(ROWS, D).astype(jnp.bfloat16)
        x_stage[...] = xbf
        xsends = []
        for d in (1, 2, 3):
            r = pltpu.make_async_remote_copy(
                src_ref=x_stage,
                dst_ref=xr.at[d - 1],
                send_sem=xssem.at[d - 1],
                recv_sem=xrsem.at[d - 1],
                device_id=(others[d - 1],),
                device_id_type=pl.DeviceIdType.MESH,
            )
            r.start()
            xsends.append(r)

        chunk_owner = [my, others[2], others[0], others[1]]
        kv_dmas = []
        for c in range(N_DEV):
            dmas = []
            bg0 = chunk_owner[c] * B_PER
            for src, dst in ((k_ref, kbuf), (v_ref, vbuf)):
                dma = pltpu.make_async_copy(
                    src.at[pl.ds(bg0, B_PER), :, pl.ds(my * D, D)],
                    dst.at[c],
                    kvsem.at[c],
                )
                dma.start()
                dmas.append(dma)
            kv_dmas.append(dmas)

        wq = (wq_ref[...] * 0.125).astype(jnp.bfloat16)
        wo = wo_ref[...].astype(jnp.bfloat16)

        def partial_half(xc, c, bi, qc):
            kb = kbuf[c, bi].astype(jnp.bfloat16)
            vb = vbuf[c, bi].astype(jnp.bfloat16)
            k3 = jnp.transpose(kb.reshape(Skv, H_LOC, DH), (1, 0, 2))
            v3 = jnp.transpose(vb.reshape(Skv, H_LOC, DH), (1, 0, 2))
            q3 = jnp.transpose(
                qc[bi * SQ:(bi + 1) * SQ].reshape(SQ, H_LOC, DH),
                (1, 0, 2))
            s = lax.dot_general(
                q3, k3, (((2,), (2,)), ((0,), (0,))),
                preferred_element_type=jnp.float32,
            )
            p = jnp.exp(s)
            l = jnp.sum(p, axis=2, keepdims=True)
            o3 = lax.dot_general(
                p.astype(jnp.bfloat16), v3,
                (((2,), (1,)), ((0,), (0,))),
                preferred_element_type=jnp.float32,
            ) * (1.0 / l)
            ao = jnp.transpose(o3, (1, 0, 2)).reshape(SQ, H_LOC * DH)
            return jnp.dot(ao.astype(jnp.bfloat16), wo,
                           preferred_element_type=jnp.float32)

        def qc_for(xc, c):
            for dma in kv_dmas[c]:
                dma.wait()
            return jnp.dot(
                xc, wq, preferred_element_type=jnp.float32
            ).astype(jnp.bfloat16)

        qc0 = qc_for(xbf, 0)
        p_own = [partial_half(xbf, 0, bi, qc0) for bi in range(B_PER)]

        psends = []
        for s in (0, 2, 1):
            rcv = pltpu.make_async_remote_copy(
                src_ref=x_stage,
                dst_ref=xr.at[s],
                send_sem=xssem.at[s],
                recv_sem=xrsem.at[s],
                device_id=(my,),
                device_id_type=pl.DeviceIdType.MESH,
            )
            rcv.wait_recv()
            d2 = 3 - s
            owner = others[d2 - 1]
            c = {0: 1, 2: 2, 1: 3}[s]
            qc = qc_for(xr[s], c)
            for bi in range(B_PER):
                pv = partial_half(xr[s], c, bi, qc)
                ps[d2 - 1, pl.ds(bi * SQ, SQ)] = pv.astype(jnp.bfloat16)
                r2 = pltpu.make_async_remote_copy(
                    src_ref=ps.at[d2 - 1, pl.ds(bi * SQ, SQ)],
                    dst_ref=pr.at[d2 - 1, pl.ds(bi * SQ, SQ)],
                    send_sem=pssem.at[d2 - 1, bi],
                    recv_sem=prsem.at[d2 - 1, bi],
                    device_id=(owner,),
                    device_id_type=pl.DeviceIdType.MESH,
                )
                r2.start()
                psends.append(r2)

        acc = jnp.concatenate(p_own, axis=0)
        for s in range(3):
            for bi in range(B_PER):
                rcv2 = pltpu.make_async_remote_copy(
                    src_ref=ps.at[s, pl.ds(bi * SQ, SQ)],
                    dst_ref=pr.at[s, pl.ds(bi * SQ, SQ)],
                    send_sem=pssem.at[s, bi],
                    recv_sem=prsem.at[s, bi],
                    device_id=(my,),
                    device_id_type=pl.DeviceIdType.MESH,
                )
                rcv2.wait_recv()
            acc = acc + pr[s].astype(jnp.float32)
        out_ref[...] = acc.reshape(B_PER, SQ, D)
        for r in xsends + psends:
            r.wait_send()

    grid_spec = pltpu.PrefetchScalarGridSpec(
        num_scalar_prefetch=1,
        grid=(1,),
        in_specs=[
            pl.BlockSpec((B_PER, SQ, D), lambda i, m: (0, 0, 0)),
            pl.BlockSpec((D, D), lambda i, m: (0, 0)),
            pl.BlockSpec((D, D), lambda i, m: (0, 0)),
            pl.BlockSpec(memory_space=pl.ANY),
            pl.BlockSpec(memory_space=pl.ANY),
        ],
        out_specs=pl.BlockSpec((B_PER, SQ, D), lambda i, m: (0, 0, 0)),
        scratch_shapes=[
            pltpu.VMEM((ROWS, D), jnp.bfloat16),
            pltpu.VMEM((N_DEV - 1, ROWS, D), jnp.bfloat16),
            pltpu.VMEM((N_DEV - 1, ROWS, D), jnp.bfloat16),
            pltpu.VMEM((N_DEV - 1, ROWS, D), jnp.bfloat16),
            pltpu.VMEM((N_DEV, B_PER, Skv, D), jnp.float32),
            pltpu.VMEM((N_DEV, B_PER, Skv, D), jnp.float32),
            pltpu.SemaphoreType.DMA((N_DEV - 1,)),
            pltpu.SemaphoreType.DMA((N_DEV - 1,)),
            pltpu.SemaphoreType.DMA((N_DEV - 1, B_PER)),
            pltpu.SemaphoreType.DMA((N_DEV - 1, B_PER)),
            pltpu.SemaphoreType.DMA((N_DEV,)),
        ],
    )

    return pl.pallas_call(
        body,
        out_shape=jax.ShapeDtypeStruct((B_PER, SQ, D), jnp.float32),
        grid_spec=grid_spec,
        compiler_params=pltpu.CompilerParams(collective_id=0),
    )(me_arr, x, Wq, Wo, K2, V2)
